# Initial kernel scaffold; baseline (speedup 1.0000x reference)
import jax
import jax.numpy as jnp
from jax.experimental import pallas as pl


def kernel(positions, hidden_states, ln1_w, ln2_w, W_in, conv_w, conv_b, dt_bias, A_log, Dparam, mamba_norm_w, W_out, W_router, W_gate_up, W_down, Ws_in, Ws_out):
    raise NotImplementedError("write your pallas kernel here")


# speedup vs baseline: 47.8555x; 47.8555x over previous
import jax
import jax.numpy as jnp
from jax.experimental import pallas as pl
from jax.experimental.pallas import tpu as pltpu

T = 2048
H = 1024
E = 8
I = 256
SHARED_I = 1024
D_INNER = 1024
DS = 32
NH = 16
HD = 64
DCONV = 4
EPS = 1e-6
CONV_DIM = D_INNER + 2 * DS

Q = 128
NC = T // Q
TB = 256
NTB = T // TB


def _rms(x, w):
    var = jnp.mean(x * x, axis=-1, keepdims=True)
    return x * jax.lax.rsqrt(var + EPS) * w


def _silu(x):
    return x * jax.nn.sigmoid(x)


def _softplus(x):
    return jnp.maximum(x, 0.0) + jnp.log1p(jnp.exp(-jnp.abs(x)))


def _mamba_body(x_ref, ln1_ref, wxbc_ref, wdt_ref, wz_ref, wout_ref,
                convw_ref, convb_ref, dtb_ref, a_ref, dfull_ref,
                eexp_ref, tri_ref, mnorm_ref, ln2_ref,
                hs2_ref, res2_ref,
                tail_ref, state_ref):
    c = pl.program_id(0)

    @pl.when(c == 0)
    def _init():
        tail_ref[...] = jnp.zeros_like(tail_ref)
        state_ref[...] = jnp.zeros_like(state_ref)

    x = x_ref[...]
    xn = _rms(x, ln1_ref[...])

    xbc_p = jnp.dot(xn, wxbc_ref[...], preferred_element_type=jnp.float32)
    dt_raw = jnp.dot(xn, wdt_ref[...], preferred_element_type=jnp.float32)
    z = jnp.dot(xn, wz_ref[...], preferred_element_type=jnp.float32)

    tail = tail_ref[...]
    acc = xbc_p * convw_ref[DCONV - 1:DCONV, :]
    for k in range(1, DCONV):
        shifted = jnp.concatenate(
            [tail[DCONV - 1 - k:, :], xbc_p[:Q - k, :]], axis=0)
        acc = acc + shifted * convw_ref[DCONV - 1 - k:DCONV - k, :]
    tail_ref[...] = xbc_p[Q - (DCONV - 1):, :]
    xbc = _silu(acc + convb_ref[...])

    x_s = xbc[:, :D_INNER]
    bm = xbc[:, D_INNER:D_INNER + DS]
    cm = xbc[:, D_INNER + DS:D_INNER + 2 * DS]

    dt = _softplus(dt_raw + dtb_ref[...])
    a = dt * a_ref[...]
    cum = jnp.dot(tri_ref[...], a,
                  preferred_element_type=jnp.float32)

    dt_e = jnp.dot(dt, eexp_ref[...], preferred_element_type=jnp.float32)
    cum_e = jnp.dot(cum, eexp_ref[...], preferred_element_type=jnp.float32)
    xd = dt_e * x_s

    mb = jax.lax.dot_general(cm, bm, (((1,), (1,)), ((), ())),
                             preferred_element_type=jnp.float32)
    row_i = jax.lax.broadcasted_iota(jnp.int32, (Q, Q), 0)
    col_i = jax.lax.broadcasted_iota(jnp.int32, (Q, Q), 1)
    causal = row_i >= col_i

    cum_t = cum.T
    ys = []
    for h in range(NH):
        colv = cum[:, h:h + 1]
        rowv = cum_t[h:h + 1, :]
        diff = jnp.where(causal, colv - rowv, -1e30)
        w_att = jnp.exp(diff) * mb
        yh = jnp.dot(w_att, xd[:, h * HD:(h + 1) * HD],
                     preferred_element_type=jnp.float32)
        ys.append(yh)
    y = jnp.concatenate(ys, axis=1)

    s_prev = state_ref[...]
    cs = jnp.dot(cm, s_prev, preferred_element_type=jnp.float32)
    y = y + jnp.exp(cum_e) * cs + dfull_ref[...] * x_s

    cum_q = cum_e[Q - 1:Q, :]
    xdec = jnp.exp(cum_q - cum_e) * xd
    state_ref[...] = jnp.exp(cum_q) * s_prev + jax.lax.dot_general(
        bm, xdec, (((0,), (0,)), ((), ())),
        preferred_element_type=jnp.float32)

    g = _rms(y * _silu(z), mnorm_ref[...])
    out = jnp.dot(g, wout_ref[...], preferred_element_type=jnp.float32)
    hs_mid = x + out
    res2_ref[...] = hs_mid
    hs2_ref[...] = _rms(hs_mid, ln2_ref[...])


def _moe_body(hs_ref, res_ref, wr_ref, wgu_ref, wdn_ref, wsi_ref, wso_ref,
              out_ref):
    hs = hs_ref[...]
    logits = jnp.dot(hs, wr_ref[...], preferred_element_type=jnp.float32)
    it = jax.lax.broadcasted_iota(jnp.int32, (TB, E), 1)
    m1 = jnp.max(logits, axis=1, keepdims=True)
    i1 = jnp.min(jnp.where(logits == m1, it, E), axis=1, keepdims=True)
    masked = jnp.where(it == i1, -1e30, logits)
    m2 = jnp.max(masked, axis=1, keepdims=True)
    i2 = jnp.min(jnp.where(masked == m2, it, E), axis=1, keepdims=True)
    w1 = jax.nn.sigmoid(m1 - m2)
    combine = (jnp.where(it == i1, w1, 0.0)
               + jnp.where(it == i2, 1.0 - w1, 0.0))

    acc = jnp.zeros((TB, H), jnp.float32)
    for e in range(E):
        gu = jnp.dot(hs, wgu_ref[e], preferred_element_type=jnp.float32)
        act = _silu(gu[:, :I]) * gu[:, I:]
        acc = acc + combine[:, e:e + 1] * jnp.dot(
            act, wdn_ref[e], preferred_element_type=jnp.float32)

    gs = jnp.dot(hs, wsi_ref[...], preferred_element_type=jnp.float32)
    acts = _silu(gs[:, :SHARED_I]) * gs[:, SHARED_I:]
    acc = acc + jnp.dot(acts, wso_ref[...], preferred_element_type=jnp.float32)
    out_ref[...] = res_ref[...] + acc


def kernel(positions, hidden_states, ln1_w, ln2_w, W_in, conv_w, conv_b,
           dt_bias, A_log, Dparam, mamba_norm_w, W_out, W_router,
           W_gate_up, W_down, Ws_in, Ws_out):
    f32 = jnp.float32
    wz = W_in[:D_INNER].T
    wxbc = W_in[D_INNER:D_INNER + CONV_DIM].T
    wdt = W_in[D_INNER + CONV_DIM:].T
    wout = W_out.T
    convw = conv_w.T
    a_neg = -jnp.exp(A_log)[None, :]
    dfull = jnp.repeat(Dparam, HD)[None, :]
    eexp = (jnp.arange(D_INNER)[None, :] // HD
            == jnp.arange(NH)[:, None]).astype(f32)
    tri = jnp.tril(jnp.ones((Q, Q), f32))

    const = lambda c: (0, 0)
    hs2, res2 = pl.pallas_call(
        _mamba_body,
        grid=(NC,),
        in_specs=[
            pl.BlockSpec((Q, H), lambda c: (c, 0)),
            pl.BlockSpec((1, H), const),
            pl.BlockSpec((H, CONV_DIM), const),
            pl.BlockSpec((H, NH), const),
            pl.BlockSpec((H, D_INNER), const),
            pl.BlockSpec((D_INNER, H), const),
            pl.BlockSpec((DCONV, CONV_DIM), const),
            pl.BlockSpec((1, CONV_DIM), const),
            pl.BlockSpec((1, NH), const),
            pl.BlockSpec((1, NH), const),
            pl.BlockSpec((1, D_INNER), const),
            pl.BlockSpec((NH, D_INNER), const),
            pl.BlockSpec((Q, Q), const),
            pl.BlockSpec((1, D_INNER), const),
            pl.BlockSpec((1, H), const),
        ],
        out_specs=[
            pl.BlockSpec((Q, H), lambda c: (c, 0)),
            pl.BlockSpec((Q, H), lambda c: (c, 0)),
        ],
        out_shape=[
            jax.ShapeDtypeStruct((T, H), f32),
            jax.ShapeDtypeStruct((T, H), f32),
        ],
        scratch_shapes=[
            pltpu.VMEM((DCONV - 1, CONV_DIM), f32),
            pltpu.VMEM((DS, D_INNER), f32),
        ],
    )(hidden_states, ln1_w[None, :], wxbc, wdt, wz, wout,
      convw, conv_b[None, :], dt_bias[None, :], a_neg, dfull,
      eexp, tri, mamba_norm_w[None, :], ln2_w[None, :])

    wr = W_router.T
    wgu = jnp.transpose(W_gate_up, (0, 2, 1))
    wdn = jnp.transpose(W_down, (0, 2, 1))
    wsi = Ws_in.T
    wso = Ws_out.T

    const3 = lambda c: (0, 0, 0)
    out = pl.pallas_call(
        _moe_body,
        grid=(NTB,),
        in_specs=[
            pl.BlockSpec((TB, H), lambda c: (c, 0)),
            pl.BlockSpec((TB, H), lambda c: (c, 0)),
            pl.BlockSpec((H, E), const),
            pl.BlockSpec((E, H, 2 * I), const3),
            pl.BlockSpec((E, I, H), const3),
            pl.BlockSpec((H, 2 * SHARED_I), const),
            pl.BlockSpec((SHARED_I, H), const),
        ],
        out_specs=pl.BlockSpec((TB, H), lambda c: (c, 0)),
        out_shape=jax.ShapeDtypeStruct((T, H), f32),
    )(hs2, res2, wr, wgu, wdn, wsi, wso)

    return (out, res2)


# speedup vs baseline: 52.8322x; 1.1040x over previous
import jax
import jax.numpy as jnp
from jax.experimental import pallas as pl
from jax.experimental.pallas import tpu as pltpu

T = 2048
H = 1024
E = 8
I = 256
SHARED_I = 1024
D_INNER = 1024
DS = 32
NH = 16
HD = 64
DCONV = 4
EPS = 1e-6
CONV_DIM = D_INNER + 2 * DS

Q = 128
NC = T // Q
TB = 256
NTB = T // TB


def _rms(x, w):
    var = jnp.mean(x * x, axis=-1, keepdims=True)
    return x * jax.lax.rsqrt(var + EPS) * w


def _silu(x):
    return x * jax.nn.sigmoid(x)


def _softplus(x):
    return jnp.maximum(x, 0.0) + jnp.log1p(jnp.exp(-jnp.abs(x)))


def _mamba_body(x_ref, ln1_ref, wxbc_ref, wdt_ref, wz_ref, wout_ref,
                convw_ref, convb_ref, dtb_ref, a_ref, dfull_ref,
                eexp_ref, tri_ref, mnorm_ref, ln2_ref,
                hs2_ref, res2_ref,
                tail_ref, state_ref):
    c = pl.program_id(0)

    @pl.when(c == 0)
    def _init():
        tail_ref[...] = jnp.zeros_like(tail_ref)
        state_ref[...] = jnp.zeros_like(state_ref)

    x = x_ref[...]
    xn = _rms(x, ln1_ref[...])
    xnb = xn.astype(jnp.bfloat16)

    xbc_p = jnp.dot(xnb, wxbc_ref[...], preferred_element_type=jnp.float32)
    dt_raw = jnp.dot(xn, wdt_ref[...], preferred_element_type=jnp.float32)
    z = jnp.dot(xnb, wz_ref[...], preferred_element_type=jnp.float32)

    tail = tail_ref[...]
    acc = xbc_p * convw_ref[DCONV - 1:DCONV, :]
    for k in range(1, DCONV):
        shifted = jnp.concatenate(
            [tail[DCONV - 1 - k:, :], xbc_p[:Q - k, :]], axis=0)
        acc = acc + shifted * convw_ref[DCONV - 1 - k:DCONV - k, :]
    tail_ref[...] = xbc_p[Q - (DCONV - 1):, :]
    xbc = _silu(acc + convb_ref[...])

    x_s = xbc[:, :D_INNER]
    bm = xbc[:, D_INNER:D_INNER + DS]
    cm = xbc[:, D_INNER + DS:D_INNER + 2 * DS]

    dt = _softplus(dt_raw + dtb_ref[...])
    a = dt * a_ref[...]
    cum = jnp.dot(tri_ref[...], a,
                  preferred_element_type=jnp.float32)

    dt_e = jnp.dot(dt, eexp_ref[...], preferred_element_type=jnp.float32)
    cum_e = jnp.dot(cum, eexp_ref[...], preferred_element_type=jnp.float32)
    xd = dt_e * x_s

    mb = jax.lax.dot_general(cm, bm, (((1,), (1,)), ((), ())),
                             preferred_element_type=jnp.float32)
    row_i = jax.lax.broadcasted_iota(jnp.int32, (Q, Q), 0)
    col_i = jax.lax.broadcasted_iota(jnp.int32, (Q, Q), 1)
    causal = row_i >= col_i

    cum_t = cum.T
    ys = []
    for h in range(NH):
        colv = cum[:, h:h + 1]
        rowv = cum_t[h:h + 1, :]
        diff = jnp.where(causal, colv - rowv, -1e30)
        w_att = jnp.exp(diff) * mb
        yh = jnp.dot(w_att, xd[:, h * HD:(h + 1) * HD],
                     preferred_element_type=jnp.float32)
        ys.append(yh)
    y = jnp.concatenate(ys, axis=1)

    s_prev = state_ref[...]
    cs = jnp.dot(cm, s_prev, preferred_element_type=jnp.float32)
    y = y + jnp.exp(cum_e) * cs + dfull_ref[...] * x_s

    cum_q = cum_e[Q - 1:Q, :]
    xdec = jnp.exp(cum_q - cum_e) * xd
    state_ref[...] = jnp.exp(cum_q) * s_prev + jax.lax.dot_general(
        bm, xdec, (((0,), (0,)), ((), ())),
        preferred_element_type=jnp.float32)

    g = _rms(y * _silu(z), mnorm_ref[...])
    out = jnp.dot(g.astype(jnp.bfloat16), wout_ref[...],
                  preferred_element_type=jnp.float32)
    hs_mid = x + out
    res2_ref[...] = hs_mid
    hs2_ref[...] = _rms(hs_mid, ln2_ref[...])


def _moe_body(hs_ref, res_ref, wr_ref, wgu_ref, wdn_ref, wsi_ref, wso_ref,
              out_ref):
    hs = hs_ref[...]
    hsb = hs.astype(jnp.bfloat16)
    logits = jnp.dot(hs, wr_ref[...], preferred_element_type=jnp.float32)
    it = jax.lax.broadcasted_iota(jnp.int32, (TB, E), 1)
    m1 = jnp.max(logits, axis=1, keepdims=True)
    i1 = jnp.min(jnp.where(logits == m1, it, E), axis=1, keepdims=True)
    masked = jnp.where(it == i1, -1e30, logits)
    m2 = jnp.max(masked, axis=1, keepdims=True)
    i2 = jnp.min(jnp.where(masked == m2, it, E), axis=1, keepdims=True)
    w1 = jax.nn.sigmoid(m1 - m2)
    combine = (jnp.where(it == i1, w1, 0.0)
               + jnp.where(it == i2, 1.0 - w1, 0.0))

    acc = jnp.zeros((TB, H), jnp.float32)
    for e in range(E):
        gu = jnp.dot(hsb, wgu_ref[e], preferred_element_type=jnp.float32)
        act = _silu(gu[:, :I]) * gu[:, I:]
        acc = acc + combine[:, e:e + 1] * jnp.dot(
            act.astype(jnp.bfloat16), wdn_ref[e],
            preferred_element_type=jnp.float32)

    gs = jnp.dot(hsb, wsi_ref[...], preferred_element_type=jnp.float32)
    acts = _silu(gs[:, :SHARED_I]) * gs[:, SHARED_I:]
    acc = acc + jnp.dot(acts.astype(jnp.bfloat16), wso_ref[...],
                        preferred_element_type=jnp.float32)
    out_ref[...] = res_ref[...] + acc


def kernel(positions, hidden_states, ln1_w, ln2_w, W_in, conv_w, conv_b,
           dt_bias, A_log, Dparam, mamba_norm_w, W_out, W_router,
           W_gate_up, W_down, Ws_in, Ws_out):
    f32 = jnp.float32
    bf16 = jnp.bfloat16
    wz = W_in[:D_INNER].T.astype(bf16)
    wxbc = W_in[D_INNER:D_INNER + CONV_DIM].T.astype(bf16)
    wdt = W_in[D_INNER + CONV_DIM:].T
    wout = W_out.T.astype(bf16)
    convw = conv_w.T
    a_neg = -jnp.exp(A_log)[None, :]
    dfull = jnp.repeat(Dparam, HD)[None, :]
    eexp = (jnp.arange(D_INNER)[None, :] // HD
            == jnp.arange(NH)[:, None]).astype(f32)
    tri = jnp.tril(jnp.ones((Q, Q), f32))

    const = lambda c: (0, 0)
    hs2, res2 = pl.pallas_call(
        _mamba_body,
        grid=(NC,),
        in_specs=[
            pl.BlockSpec((Q, H), lambda c: (c, 0)),
            pl.BlockSpec((1, H), const),
            pl.BlockSpec((H, CONV_DIM), const),
            pl.BlockSpec((H, NH), const),
            pl.BlockSpec((H, D_INNER), const),
            pl.BlockSpec((D_INNER, H), const),
            pl.BlockSpec((DCONV, CONV_DIM), const),
            pl.BlockSpec((1, CONV_DIM), const),
            pl.BlockSpec((1, NH), const),
            pl.BlockSpec((1, NH), const),
            pl.BlockSpec((1, D_INNER), const),
            pl.BlockSpec((NH, D_INNER), const),
            pl.BlockSpec((Q, Q), const),
            pl.BlockSpec((1, D_INNER), const),
            pl.BlockSpec((1, H), const),
        ],
        out_specs=[
            pl.BlockSpec((Q, H), lambda c: (c, 0)),
            pl.BlockSpec((Q, H), lambda c: (c, 0)),
        ],
        out_shape=[
            jax.ShapeDtypeStruct((T, H), f32),
            jax.ShapeDtypeStruct((T, H), f32),
        ],
        scratch_shapes=[
            pltpu.VMEM((DCONV - 1, CONV_DIM), f32),
            pltpu.VMEM((DS, D_INNER), f32),
        ],
    )(hidden_states, ln1_w[None, :], wxbc, wdt, wz, wout,
      convw, conv_b[None, :], dt_bias[None, :], a_neg, dfull,
      eexp, tri, mamba_norm_w[None, :], ln2_w[None, :])

    wr = W_router.T
    wgu = jnp.transpose(W_gate_up, (0, 2, 1)).astype(bf16)
    wdn = jnp.transpose(W_down, (0, 2, 1)).astype(bf16)
    wsi = Ws_in.T.astype(bf16)
    wso = Ws_out.T.astype(bf16)

    const3 = lambda c: (0, 0, 0)
    out = pl.pallas_call(
        _moe_body,
        grid=(NTB,),
        in_specs=[
            pl.BlockSpec((TB, H), lambda c: (c, 0)),
            pl.BlockSpec((TB, H), lambda c: (c, 0)),
            pl.BlockSpec((H, E), const),
            pl.BlockSpec((E, H, 2 * I), const3),
            pl.BlockSpec((E, I, H), const3),
            pl.BlockSpec((H, 2 * SHARED_I), const),
            pl.BlockSpec((SHARED_I, H), const),
        ],
        out_specs=pl.BlockSpec((TB, H), lambda c: (c, 0)),
        out_shape=jax.ShapeDtypeStruct((T, H), f32),
    )(hs2, res2, wr, wgu, wdn, wsi, wso)

    return (out, res2)


# speedup vs baseline: 55.3919x; 1.0484x over previous
import jax
import jax.numpy as jnp
from jax.experimental import pallas as pl
from jax.experimental.pallas import tpu as pltpu

T = 2048
H = 1024
E = 8
I = 256
SHARED_I = 1024
D_INNER = 1024
DS = 32
NH = 16
HD = 64
DCONV = 4
EPS = 1e-6
CONV_DIM = D_INNER + 2 * DS
ZXBC = D_INNER + CONV_DIM

Q = 256
NC = T // Q


def _rms(x, w):
    var = jnp.mean(x * x, axis=-1, keepdims=True)
    return x * jax.lax.rsqrt(var + EPS) * w


def _silu(x):
    return x * jax.nn.sigmoid(x)


def _softplus(x):
    return jnp.maximum(x, 0.0) + jnp.log1p(jnp.exp(-jnp.abs(x)))


def _body(x_ref, ln1_ref, ln2_ref, mnorm_ref,
          wzxbc_ref, wdt_ref, wout_ref,
          convw_ref, convb_ref, dtb_ref, a_ref, dfull_ref,
          eexp_ref, tri_ref,
          wr_ref, wgu_ref, wdn_ref, wsi_ref, wso_ref,
          out_ref, res2_ref,
          tail_ref, state_ref):
    c = pl.program_id(0)

    @pl.when(c == 0)
    def _init():
        tail_ref[...] = jnp.zeros_like(tail_ref)
        state_ref[...] = jnp.zeros_like(state_ref)

    x = x_ref[...]
    xn = _rms(x, ln1_ref[...])
    xnb = xn.astype(jnp.bfloat16)

    p = jnp.dot(xnb, wzxbc_ref[...], preferred_element_type=jnp.float32)
    z = p[:, :D_INNER]
    xbc_p = p[:, D_INNER:]
    dt_raw = jnp.dot(xn, wdt_ref[...], preferred_element_type=jnp.float32)

    tail = tail_ref[...]
    acc = xbc_p * convw_ref[DCONV - 1:DCONV, :]
    for k in range(1, DCONV):
        shifted = jnp.concatenate(
            [tail[DCONV - 1 - k:, :], xbc_p[:Q - k, :]], axis=0)
        acc = acc + shifted * convw_ref[DCONV - 1 - k:DCONV - k, :]
    tail_ref[...] = xbc_p[Q - (DCONV - 1):, :]
    xbc = _silu(acc + convb_ref[...])

    x_s = xbc[:, :D_INNER]
    bm = xbc[:, D_INNER:D_INNER + DS]
    cm = xbc[:, D_INNER + DS:D_INNER + 2 * DS]

    dt = _softplus(dt_raw + dtb_ref[...])
    a = dt * a_ref[...]
    cum = jnp.dot(tri_ref[...], a,
                  preferred_element_type=jnp.float32)

    dt_e = jnp.dot(dt, eexp_ref[...], preferred_element_type=jnp.float32)
    cum_e = jnp.dot(cum, eexp_ref[...], preferred_element_type=jnp.float32)
    xd = dt_e * x_s

    mb = jax.lax.dot_general(cm, bm, (((1,), (1,)), ((), ())),
                             preferred_element_type=jnp.float32)
    row_i = jax.lax.broadcasted_iota(jnp.int32, (Q, Q), 0)
    col_i = jax.lax.broadcasted_iota(jnp.int32, (Q, Q), 1)
    causal = row_i >= col_i

    cum_t = cum.T
    ys = []
    for h in range(NH):
        diff = jnp.where(causal, cum[:, h:h + 1] - cum_t[h:h + 1, :], -1e30)
        w_att = jnp.exp(diff) * mb
        yh = jnp.dot(w_att, xd[:, h * HD:(h + 1) * HD],
                     preferred_element_type=jnp.float32)
        ys.append(yh)
    y = jnp.concatenate(ys, axis=1)

    s_prev = state_ref[...]
    cs = jnp.dot(cm, s_prev, preferred_element_type=jnp.float32)
    y = y + jnp.exp(cum_e) * cs + dfull_ref[...] * x_s

    cum_q = cum_e[Q - 1:Q, :]
    xdec = jnp.exp(cum_q - cum_e) * xd
    state_ref[...] = jnp.exp(cum_q) * s_prev + jax.lax.dot_general(
        bm, xdec, (((0,), (0,)), ((), ())),
        preferred_element_type=jnp.float32)

    g = _rms(y * _silu(z), mnorm_ref[...])
    out_m = jnp.dot(g.astype(jnp.bfloat16), wout_ref[...],
                    preferred_element_type=jnp.float32)
    hs_mid = x + out_m
    res2_ref[...] = hs_mid
    hs2 = _rms(hs_mid, ln2_ref[...])
    hs2b = hs2.astype(jnp.bfloat16)

    logits = jnp.dot(hs2, wr_ref[...], preferred_element_type=jnp.float32)
    it = jax.lax.broadcasted_iota(jnp.int32, (Q, E), 1)
    m1 = jnp.max(logits, axis=1, keepdims=True)
    i1 = jnp.min(jnp.where(logits == m1, it, E), axis=1, keepdims=True)
    masked = jnp.where(it == i1, -1e30, logits)
    m2 = jnp.max(masked, axis=1, keepdims=True)
    i2 = jnp.min(jnp.where(masked == m2, it, E), axis=1, keepdims=True)
    w1 = jax.nn.sigmoid(m1 - m2)
    combine = (jnp.where(it == i1, w1, 0.0)
               + jnp.where(it == i2, 1.0 - w1, 0.0))

    acc2 = jnp.zeros((Q, H), jnp.float32)
    for e in range(E):
        gu = jnp.dot(hs2b, wgu_ref[e], preferred_element_type=jnp.float32)
        act = _silu(gu[:, :I]) * gu[:, I:]
        acc2 = acc2 + combine[:, e:e + 1] * jnp.dot(
            act.astype(jnp.bfloat16), wdn_ref[e],
            preferred_element_type=jnp.float32)
    gs = jnp.dot(hs2b, wsi_ref[...], preferred_element_type=jnp.float32)
    acts = _silu(gs[:, :SHARED_I]) * gs[:, SHARED_I:]
    acc2 = acc2 + jnp.dot(acts.astype(jnp.bfloat16), wso_ref[...],
                          preferred_element_type=jnp.float32)
    out_ref[...] = hs_mid + acc2


def kernel(positions, hidden_states, ln1_w, ln2_w, W_in, conv_w, conv_b,
           dt_bias, A_log, Dparam, mamba_norm_w, W_out, W_router,
           W_gate_up, W_down, Ws_in, Ws_out):
    f32 = jnp.float32
    bf16 = jnp.bfloat16
    wzxbc = W_in[:ZXBC].T.astype(bf16)
    wdt = W_in[ZXBC:].T
    wout = W_out.T.astype(bf16)
    wgu = jnp.transpose(W_gate_up, (0, 2, 1)).astype(bf16)
    wdn = jnp.transpose(W_down, (0, 2, 1)).astype(bf16)
    wsi = Ws_in.T.astype(bf16)
    wso = Ws_out.T.astype(bf16)
    wr = W_router.T
    convw = conv_w.T
    a_neg = -jnp.exp(A_log)[None, :]
    dfull = jnp.repeat(Dparam, HD)[None, :]
    eexp = (jnp.arange(D_INNER)[None, :] // HD
            == jnp.arange(NH)[:, None]).astype(f32)
    tri = (jnp.arange(Q)[:, None]
           >= jnp.arange(Q)[None, :]).astype(f32)

    const = lambda c: (0, 0)
    const3 = lambda c: (0, 0, 0)
    out, res2 = pl.pallas_call(
        _body,
        grid=(NC,),
        in_specs=[
            pl.BlockSpec((Q, H), lambda c: (c, 0)),
            pl.BlockSpec((1, H), const),
            pl.BlockSpec((1, H), const),
            pl.BlockSpec((1, D_INNER), const),
            pl.BlockSpec((H, ZXBC), const),
            pl.BlockSpec((H, NH), const),
            pl.BlockSpec((D_INNER, H), const),
            pl.BlockSpec((DCONV, CONV_DIM), const),
            pl.BlockSpec((1, CONV_DIM), const),
            pl.BlockSpec((1, NH), const),
            pl.BlockSpec((1, NH), const),
            pl.BlockSpec((1, D_INNER), const),
            pl.BlockSpec((NH, D_INNER), const),
            pl.BlockSpec((Q, Q), const),
            pl.BlockSpec((H, E), const),
            pl.BlockSpec((E, H, 2 * I), const3),
            pl.BlockSpec((E, I, H), const3),
            pl.BlockSpec((H, 2 * SHARED_I), const),
            pl.BlockSpec((SHARED_I, H), const),
        ],
        out_specs=[
            pl.BlockSpec((Q, H), lambda c: (c, 0)),
            pl.BlockSpec((Q, H), lambda c: (c, 0)),
        ],
        out_shape=[
            jax.ShapeDtypeStruct((T, H), f32),
            jax.ShapeDtypeStruct((T, H), f32),
        ],
        scratch_shapes=[
            pltpu.VMEM((DCONV - 1, CONV_DIM), f32),
            pltpu.VMEM((DS, D_INNER), f32),
        ],
    )(hidden_states, ln1_w[None, :], ln2_w[None, :], mamba_norm_w[None, :],
      wzxbc, wdt, wout, convw, conv_b[None, :], dt_bias[None, :], a_neg,
      dfull, eexp, tri, wr, wgu, wdn, wsi, wso)

    return (out, res2)
